# baseline (speedup 1.0000x reference)
import functools

import jax
import jax.numpy as jnp
from jax import lax
from jax.experimental import pallas as pl
from jax.experimental.pallas import tpu as pltpu
from jax.experimental.pallas import tpu_sc as plsc

B = 1024
D = 1024
DFF = 1024
E = 8
K = 2
GH = 512
CH = 512
OUT = 3

NW = 32
CHUNK = B // NW


def _gating_body(x_ref, Wg1_ref, bg1_ref, Wg2_ref, bg2_ref,
                 probs_ref, probsT_ref):
    h = jnp.maximum(
        jnp.dot(x_ref[...], Wg1_ref[...], preferred_element_type=jnp.float32)
        + bg1_ref[...],
        0.0,
    )
    logits = (
        jnp.dot(h, Wg2_ref[...], preferred_element_type=jnp.float32)
        + bg2_ref[...]
    )
    m = jnp.max(logits, axis=1, keepdims=True)
    ex = jnp.exp(logits - m)
    probs = ex / jnp.sum(ex, axis=1, keepdims=True)
    probs_ref[...] = probs
    probsT_ref[...] = jnp.transpose(probs)


@functools.cache
def _get_route():
    mesh = plsc.VectorSubcoreMesh(core_axis_name="c", subcore_axis_name="s",
                                  num_cores=1)
    RCHUNK = B // 16

    @functools.partial(
        pl.kernel,
        out_type=jax.ShapeDtypeStruct((E * B,), jnp.float32),
        mesh=mesh,
        scratch_types=[
            pltpu.VMEM((E, RCHUNK), jnp.float32),
            pltpu.VMEM((E, RCHUNK), jnp.float32),
            pltpu.SemaphoreType.DMA,
            pltpu.SemaphoreType.DMA,
        ],
    )
    def _route(probsT_hbm, wt_hbm, pv_v, wt_v, sem, osem):
        wid = lax.axis_index("s") + lax.axis_index("c") * 16
        base = wid * RCHUNK
        cps = [
            pltpu.async_copy(probsT_hbm.at[pl.ds(e * B + base, RCHUNK)],
                             pv_v.at[e], sem)
            for e in range(E)
        ]
        for cp in cps:
            cp.wait()
        for h in range(RCHUNK // 16):
            sl = pl.ds(h * 16, 16)
            pe = [pv_v[e, sl] for e in range(E)]
            m1 = pe[0]
            for e in range(1, E):
                m1 = jnp.maximum(m1, pe[e])
            i1 = jnp.full((16,), E, jnp.int32)
            for e in range(E - 1, -1, -1):
                i1 = jnp.where(pe[e] == m1, e, i1)
            pm = [jnp.where(i1 == e, -1.0, pe[e]) for e in range(E)]
            m2 = pm[0]
            for e in range(1, E):
                m2 = jnp.maximum(m2, pm[e])
            i2 = jnp.full((16,), E, jnp.int32)
            for e in range(E - 1, -1, -1):
                i2 = jnp.where(pm[e] == m2, e, i2)
            for e in range(E):
                wt_v[e, sl] = (jnp.where(i1 == e, m1, 0.0)
                               + jnp.where(i2 == e, m2, 0.0))
        ocps = [
            pltpu.async_copy(wt_v.at[e],
                             wt_hbm.at[pl.ds(e * B + base, RCHUNK)], osem)
            for e in range(E)
        ]
        for cp in ocps:
            cp.wait()

    return _route


def _moe_body(x_ref, We1_ref, be1_ref, We2_ref, be2_ref, wT_ref,
              Wc1_ref, bc1_ref, Wc2_ref, bc2_ref, out_ref, acc_ref):
    e = pl.program_id(0)
    j = pl.program_id(1)
    h = jnp.maximum(
        jnp.dot(x_ref[...], We1_ref[0], preferred_element_type=jnp.float32)
        + be1_ref[0],
        0.0,
    )
    y = jnp.dot(h, We2_ref[0], preferred_element_type=jnp.float32)
    y = jnp.where(j == 0, y + be2_ref[0], y)
    ee = lax.broadcasted_iota(jnp.int32, (E, B), 0)
    row = jnp.sum(jnp.where(ee == e, wT_ref[...], 0.0), axis=0,
                  keepdims=True)
    contrib = jnp.transpose(row) * y
    first = (e == 0) & (j == 0)

    @pl.when(first)
    def _():
        acc_ref[...] = contrib

    @pl.when(jnp.logical_not(first))
    def _():
        acc_ref[...] += contrib

    @pl.when((e == E - 1) & (j == 1))
    def _():
        hc = jnp.maximum(
            jnp.dot(acc_ref[...], Wc1_ref[...],
                    preferred_element_type=jnp.float32)
            + bc1_ref[...],
            0.0,
        )
        out_ref[...] = (
            jnp.dot(hc, Wc2_ref[...], preferred_element_type=jnp.float32)
            + bc2_ref[...]
        )


def kernel(x, Wg1, bg1, Wg2, bg2, We1, be1, We2, be2, Wc1, bc1, Wc2, bc2):
    probs, probsT = pl.pallas_call(
        _gating_body,
        out_shape=(
            jax.ShapeDtypeStruct((B, E), jnp.float32),
            jax.ShapeDtypeStruct((E, B), jnp.float32),
        ),
    )(x, Wg1, bg1.reshape(1, GH), Wg2, bg2.reshape(1, E))

    wT = _get_route()(probsT.reshape(E * B)).reshape(E, B)

    logits = pl.pallas_call(
        _moe_body,
        grid=(E, 2),
        in_specs=[
            pl.BlockSpec((B, D), lambda e, j: (0, 0)),
            pl.BlockSpec((1, D, DFF // 2), lambda e, j: (e, 0, j)),
            pl.BlockSpec((1, 1, DFF // 2), lambda e, j: (e, 0, j)),
            pl.BlockSpec((1, DFF // 2, D), lambda e, j: (e, j, 0)),
            pl.BlockSpec((1, 1, D), lambda e, j: (e, 0, 0)),
            pl.BlockSpec((E, B), lambda e, j: (0, 0)),
            pl.BlockSpec((D, CH), lambda e, j: (0, 0)),
            pl.BlockSpec((1, CH), lambda e, j: (0, 0)),
            pl.BlockSpec((CH, OUT), lambda e, j: (0, 0)),
            pl.BlockSpec((1, OUT), lambda e, j: (0, 0)),
        ],
        out_specs=pl.BlockSpec((B, OUT), lambda e, j: (0, 0)),
        out_shape=jax.ShapeDtypeStruct((B, OUT), jnp.float32),
        scratch_shapes=[pltpu.VMEM((B, D), jnp.float32)],
    )(x, We1, be1.reshape(E, 1, DFF), We2, be2.reshape(E, 1, D), wT,
      Wc1, bc1.reshape(1, CH), Wc2, bc2.reshape(1, OUT))

    return (logits, probs)


# speedup vs baseline: 1.0528x; 1.0528x over previous
import functools

import jax
import jax.numpy as jnp
from jax import lax
from jax.experimental import pallas as pl
from jax.experimental.pallas import tpu as pltpu
from jax.experimental.pallas import tpu_sc as plsc

B = 1024
D = 1024
DFF = 1024
E = 8
K = 2
GH = 512
CH = 512
OUT = 3

NW = 32
CHUNK = B // NW


def _gating_body(x_ref, Wg1_ref, bg1_ref, Wg2_ref, bg2_ref,
                 probs_ref, probsT_ref):
    h = jnp.maximum(
        jnp.dot(x_ref[...], Wg1_ref[...], preferred_element_type=jnp.float32)
        + bg1_ref[...],
        0.0,
    )
    logits = (
        jnp.dot(h, Wg2_ref[...], preferred_element_type=jnp.float32)
        + bg2_ref[...]
    )
    m = jnp.max(logits, axis=1, keepdims=True)
    ex = jnp.exp(logits - m)
    probs = ex / jnp.sum(ex, axis=1, keepdims=True)
    probs_ref[...] = probs
    probsT_ref[...] = jnp.transpose(probs)


@functools.cache
def _get_route():
    mesh = plsc.VectorSubcoreMesh(core_axis_name="c", subcore_axis_name="s",
                                  num_cores=1)
    RCHUNK = B // 16

    @functools.partial(
        pl.kernel,
        out_type=jax.ShapeDtypeStruct((E * B,), jnp.float32),
        mesh=mesh,
        scratch_types=[
            pltpu.VMEM((E, RCHUNK), jnp.float32),
            pltpu.VMEM((E, RCHUNK), jnp.float32),
            pltpu.SemaphoreType.DMA,
            pltpu.SemaphoreType.DMA,
        ],
    )
    def _route(probsT_hbm, wt_hbm, pv_v, wt_v, sem, osem):
        wid = lax.axis_index("s") + lax.axis_index("c") * 16
        base = wid * RCHUNK
        cps = [
            pltpu.async_copy(probsT_hbm.at[pl.ds(e * B + base, RCHUNK)],
                             pv_v.at[e], sem)
            for e in range(E)
        ]
        for cp in cps:
            cp.wait()
        for h in range(RCHUNK // 16):
            sl = pl.ds(h * 16, 16)
            pe = [pv_v[e, sl] for e in range(E)]
            m1 = pe[0]
            for e in range(1, E):
                m1 = jnp.maximum(m1, pe[e])
            i1 = jnp.full((16,), E, jnp.int32)
            for e in range(E - 1, -1, -1):
                i1 = jnp.where(pe[e] == m1, e, i1)
            pm = [jnp.where(i1 == e, -1.0, pe[e]) for e in range(E)]
            m2 = pm[0]
            for e in range(1, E):
                m2 = jnp.maximum(m2, pm[e])
            i2 = jnp.full((16,), E, jnp.int32)
            for e in range(E - 1, -1, -1):
                i2 = jnp.where(pm[e] == m2, e, i2)
            for e in range(E):
                wt_v[e, sl] = (jnp.where(i1 == e, m1, 0.0)
                               + jnp.where(i2 == e, m2, 0.0))
        ocps = [
            pltpu.async_copy(wt_v.at[e],
                             wt_hbm.at[pl.ds(e * B + base, RCHUNK)], osem)
            for e in range(E)
        ]
        for cp in ocps:
            cp.wait()

    return _route


def _moe_body(x_ref, We1_ref, be1_ref, We2_ref, be2_ref, wT_ref,
              Wc1_ref, bc1_ref, Wc2_ref, bc2_ref, out_ref, acc_ref):
    e = pl.program_id(0)
    h = jnp.maximum(
        jnp.dot(x_ref[...], We1_ref[0], preferred_element_type=jnp.float32)
        + be1_ref[0],
        0.0,
    )
    y = (
        jnp.dot(h, We2_ref[0], preferred_element_type=jnp.float32)
        + be2_ref[0]
    )
    ee = lax.broadcasted_iota(jnp.int32, (E, B), 0)
    row = jnp.sum(jnp.where(ee == e, wT_ref[...], 0.0), axis=0,
                  keepdims=True)
    contrib = jnp.transpose(row) * y

    @pl.when(e == 0)
    def _():
        acc_ref[...] = contrib

    @pl.when(e != 0)
    def _():
        acc_ref[...] += contrib

    @pl.when(e == E - 1)
    def _():
        hc = jnp.maximum(
            jnp.dot(acc_ref[...], Wc1_ref[...],
                    preferred_element_type=jnp.float32)
            + bc1_ref[...],
            0.0,
        )
        out_ref[...] = (
            jnp.dot(hc, Wc2_ref[...], preferred_element_type=jnp.float32)
            + bc2_ref[...]
        )


def kernel(x, Wg1, bg1, Wg2, bg2, We1, be1, We2, be2, Wc1, bc1, Wc2, bc2):
    probs, probsT = pl.pallas_call(
        _gating_body,
        out_shape=(
            jax.ShapeDtypeStruct((B, E), jnp.float32),
            jax.ShapeDtypeStruct((E, B), jnp.float32),
        ),
    )(x, Wg1, bg1.reshape(1, GH), Wg2, bg2.reshape(1, E))

    wT = _get_route()(probsT.reshape(E * B)).reshape(E, B)

    logits = pl.pallas_call(
        _moe_body,
        grid=(E,),
        in_specs=[
            pl.BlockSpec((B, D), lambda e: (0, 0)),
            pl.BlockSpec((1, D, DFF), lambda e: (e, 0, 0)),
            pl.BlockSpec((1, 1, DFF), lambda e: (e, 0, 0)),
            pl.BlockSpec((1, DFF, D), lambda e: (e, 0, 0)),
            pl.BlockSpec((1, 1, D), lambda e: (e, 0, 0)),
            pl.BlockSpec((E, B), lambda e: (0, 0)),
            pl.BlockSpec((D, CH), lambda e: (0, 0)),
            pl.BlockSpec((1, CH), lambda e: (0, 0)),
            pl.BlockSpec((CH, OUT), lambda e: (0, 0)),
            pl.BlockSpec((1, OUT), lambda e: (0, 0)),
        ],
        out_specs=pl.BlockSpec((B, OUT), lambda e: (0, 0)),
        out_shape=jax.ShapeDtypeStruct((B, OUT), jnp.float32),
        scratch_shapes=[pltpu.VMEM((B, D), jnp.float32)],
    )(x, We1, be1.reshape(E, 1, DFF), We2, be2.reshape(E, 1, D), wT,
      Wc1, bc1.reshape(1, CH), Wc2, bc2.reshape(1, OUT))

    return (logits, probs)
